# Initial kernel scaffold; baseline (speedup 1.0000x reference)
import jax
import jax.numpy as jnp
from jax.experimental import pallas as pl


def kernel(x, edge_index, batch, W1s, b1s, gammas, betas, W2s, b2s, epss, Wh1, bh1, Wh2, bh2):
    raise NotImplementedError("write your pallas kernel here")


# speedup vs baseline: 3.5387x; 3.5387x over previous
import functools

import jax
import jax.numpy as jnp
from jax import lax
from jax.experimental import pallas as pl
from jax.experimental.pallas import tpu as pltpu
from jax.experimental.pallas import tpu_sc as plsc

D = 128
NUM_WORKERS = 32
CHUNK = 128


@functools.partial(jax.jit, static_argnames=("n_pad", "e_w"))
def _sc_agg(h, src, dst, zeros128, *, n_pad, e_w):
    n_chunks = e_w // CHUNK
    rows_per_tile = n_pad // 16
    mesh = plsc.VectorSubcoreMesh(core_axis_name="c", subcore_axis_name="s")

    @functools.partial(
        pl.kernel,
        mesh=mesh,
        out_type=jax.ShapeDtypeStruct((2, n_pad, D), jnp.float32),
        scratch_types=[
            pltpu.VMEM((CHUNK,), jnp.int32),
            pltpu.VMEM((CHUNK,), jnp.int32),
            pltpu.VMEM((CHUNK, D), jnp.float32),
            pltpu.VMEM_SHARED((n_pad, D), jnp.float32),
            pltpu.SemaphoreType.DMA,
        ],
    )
    def agg_kernel(h_hbm, src_hbm, dst_hbm, z_hbm, out_hbm,
                   src_v, dst_v, rows_v, acc, sem):
        c = lax.axis_index("c")
        s = lax.axis_index("s")
        w = s * 2 + c
        for j in range(rows_per_tile // 128):
            pltpu.sync_copy(z_hbm, acc.at[pl.ds(s * rows_per_tile + j * 128, 128)])
        plsc.subcore_barrier()
        base = w * e_w

        def body(k, carry):
            off = base + k * CHUNK
            pltpu.sync_copy(src_hbm.at[pl.ds(off, CHUNK)], src_v)
            pltpu.sync_copy(dst_hbm.at[pl.ds(off, CHUNK)], dst_v)
            pltpu.async_copy(h_hbm.at[src_v], rows_v, sem).wait()
            pltpu.sync_copy(rows_v, acc.at[dst_v], add=True)
            return carry

        lax.fori_loop(0, n_chunks, body, 0)
        plsc.subcore_barrier()
        for j in range(rows_per_tile // 128):
            r0 = s * rows_per_tile + j * 128
            pltpu.sync_copy(acc.at[pl.ds(r0, 128)], out_hbm.at[c, pl.ds(r0, 128)])

    return agg_kernel(h, src, dst, zeros128)


@jax.jit
def _sc_pool(h, batch_pad, zeros8, ones64):
    n_pad = h.shape[0]
    per_w = n_pad // NUM_WORKERS
    pchunk = 64
    n_chunks = per_w // pchunk
    mesh = plsc.VectorSubcoreMesh(core_axis_name="c", subcore_axis_name="s")

    @functools.partial(
        pl.kernel,
        mesh=mesh,
        out_type=(
            jax.ShapeDtypeStruct((2, 128, D), jnp.float32),
            jax.ShapeDtypeStruct((2, 128, D), jnp.float32),
        ),
        scratch_types=[
            pltpu.VMEM((pchunk,), jnp.int32),
            pltpu.VMEM((pchunk, D), jnp.float32),
            pltpu.VMEM((pchunk, D), jnp.float32),
            pltpu.VMEM_SHARED((128, D), jnp.float32),
            pltpu.VMEM_SHARED((128, D), jnp.float32),
        ],
    )
    def pool_kernel(h_hbm, b_hbm, z_hbm, ones_hbm, sums_hbm, cnt_hbm,
                    idx_v, rows_v, ones_v, acc_s, acc_c):
        c = lax.axis_index("c")
        s = lax.axis_index("s")
        w = s * 2 + c
        pltpu.sync_copy(z_hbm, acc_s.at[pl.ds(s * 8, 8)])
        pltpu.sync_copy(z_hbm, acc_c.at[pl.ds(s * 8, 8)])
        pltpu.sync_copy(ones_hbm, ones_v)
        plsc.subcore_barrier()
        base = w * per_w
        for k in range(n_chunks):
            off = base + k * pchunk
            pltpu.sync_copy(b_hbm.at[pl.ds(off, pchunk)], idx_v)
            pltpu.sync_copy(h_hbm.at[pl.ds(off, pchunk)], rows_v)
            pltpu.sync_copy(rows_v, acc_s.at[idx_v], add=True)
            pltpu.sync_copy(ones_v, acc_c.at[idx_v], add=True)
        plsc.subcore_barrier()
        pltpu.sync_copy(acc_s.at[pl.ds(s * 8, 8)], sums_hbm.at[c, pl.ds(s * 8, 8)])
        pltpu.sync_copy(acc_c.at[pl.ds(s * 8, 8)], cnt_hbm.at[c, pl.ds(s * 8, 8)])

    return pool_kernel(h, batch_pad, zeros8, ones64)


def _mlp_body(eps_ref, h_ref, p0_ref, p1_ref, w1_ref, b1_ref, w2_ref, b2_ref,
              out_ref):
    z = h_ref[...] * eps_ref[0] + p0_ref[...] + p1_ref[...]
    z = jnp.dot(z, w1_ref[...], preferred_element_type=jnp.float32) + b1_ref[...]
    z = jnp.maximum(z, 0.0)
    z = jnp.dot(z, w2_ref[...], preferred_element_type=jnp.float32) + b2_ref[...]
    out_ref[...] = jnp.maximum(z, 0.0)


def _tc_mlp(h, p0, p1, w1, b1, w2, b2, eps1, *, blk=1024):
    n_pad = h.shape[0]
    grid = (n_pad // blk,)
    row_spec = pl.BlockSpec((blk, D), lambda i: (i, 0))
    full = pl.BlockSpec((D, D), lambda i: (0, 0))
    bias = pl.BlockSpec((1, D), lambda i: (0, 0))
    return pl.pallas_call(
        _mlp_body,
        grid=grid,
        in_specs=[
            pl.BlockSpec(memory_space=pltpu.SMEM),
            row_spec, row_spec, row_spec, full, bias, full, bias,
        ],
        out_specs=row_spec,
        out_shape=jax.ShapeDtypeStruct((n_pad, D), jnp.float32),
    )(eps1, h, p0, p1, w1, b1, w2, b2)


def _head_body(sums_ref, cnt_ref, wh1_ref, bh1_ref, wh2_ref, bh2_ref, out_ref):
    sums = sums_ref[0] + sums_ref[1]
    cnt = cnt_ref[0] + cnt_ref[1]
    pooled = (sums / jnp.maximum(cnt, 1.0))[:64]
    t = jnp.dot(pooled, wh1_ref[...], preferred_element_type=jnp.float32)
    t = jnp.maximum(t + bh1_ref[...], 0.0)
    out_ref[...] = (
        jnp.dot(t, wh2_ref[...], preferred_element_type=jnp.float32) + bh2_ref[...]
    )


def _tc_head(sums, cnt, wh1, bh1, wh2, bh2, n_classes):
    return pl.pallas_call(
        _head_body,
        out_shape=jax.ShapeDtypeStruct((64, n_classes), jnp.float32),
    )(sums, cnt, wh1, bh1, wh2, bh2)


def kernel(x, edge_index, batch, W1s, b1s, gammas, betas, W2s, b2s, epss,
           Wh1, bh1, Wh2, bh2):
    n, _ = x.shape
    e = edge_index.shape[1]
    n_layers = W1s.shape[0]
    n_classes = Wh2.shape[1]

    n_pad = ((n + 1023) // 1024) * 1024
    e_w = ((e + NUM_WORKERS * CHUNK - 1) // (NUM_WORKERS * CHUNK)) * CHUNK
    e_pad = NUM_WORKERS * e_w

    h = jnp.pad(x, ((0, n_pad - n), (0, 0)))
    src = jnp.pad(edge_index[0], (0, e_pad - e))
    dst = jnp.pad(edge_index[1], (0, e_pad - e), constant_values=n)
    batch_pad = jnp.pad(batch, (0, n_pad - n), constant_values=64)

    zeros128 = jnp.zeros((128, D), jnp.float32)
    zeros8 = jnp.zeros((8, D), jnp.float32)
    ones64 = jnp.ones((64, D), jnp.float32)

    bn_scale = 1.0 / jnp.sqrt(1.0 + 1e-5)

    for i in range(n_layers):
        parts = _sc_agg(h, src, dst, zeros128, n_pad=n_pad, e_w=e_w)
        g = bn_scale * gammas[i]
        w1 = W1s[i] * g[None, :]
        b1 = (b1s[i] * g + betas[i]).reshape(1, D)
        eps1 = jnp.reshape(1.0 + epss[i], (1,))
        h = _tc_mlp(h, parts[0], parts[1], w1, b1, W2s[i],
                    b2s[i].reshape(1, D), eps1)

    sums_p, cnt_p = _sc_pool(h, batch_pad, zeros8, ones64)
    return _tc_head(sums_p, cnt_p, Wh1, bh1.reshape(1, -1), Wh2,
                    bh2.reshape(1, -1), n_classes)
